# baseline (speedup 1.0000x reference)
import functools

import jax
import jax.numpy as jnp
from jax import lax
from jax.experimental import pallas as pl
from jax.experimental.pallas import tpu as pltpu
from jax.experimental.pallas import tpu_sc as plsc

B = 16384
DIM = 64
N_OUT = 1000

_NC = 2
_NS = 16
_NW = _NC * _NS
_BPW = B // _NW
_CHUNK = 128
_NCHUNK = _BPW // _CHUNK

_VB = 8192
_H = 31 * _VB
_HDIM = DIM // 2


def _pack(t_f32):
    tb = t_f32.astype(jnp.bfloat16)
    lo = lax.bitcast_convert_type(tb[:, :_HDIM], jnp.uint16).astype(jnp.uint32)
    hi = lax.bitcast_convert_type(tb[:, _HDIM:], jnp.uint16).astype(jnp.uint32)
    return lax.bitcast_convert_type(lo | (hi << 16), jnp.int32)


def _conv_body(t0_ref, t1_ref, t2_ref, t3_ref, out_ref):
    out_ref[:, 0 * _HDIM:1 * _HDIM] = _pack(t0_ref[...].T)
    out_ref[:, 1 * _HDIM:2 * _HDIM] = _pack(t1_ref[...].T)
    out_ref[:, 2 * _HDIM:3 * _HDIM] = _pack(t2_ref[...].T)
    out_ref[:, 3 * _HDIM:4 * _HDIM] = _pack(t3_ref[...].T)


def _tc_convert(tt):
    grid = (31,)
    return pl.pallas_call(
        _conv_body,
        grid=grid,
        in_specs=[
            pl.BlockSpec((DIM, _VB), lambda i: (0, i)),
            pl.BlockSpec((DIM, _VB), lambda i: (0, i + 31)),
            pl.BlockSpec((DIM, _VB), lambda i: (0, i + 62)),
            pl.BlockSpec((DIM, _VB), lambda i: (0, jnp.minimum(i + 93, 122))),
        ],
        out_specs=pl.BlockSpec((_VB, 4 * _HDIM), lambda i: (i, 0)),
        out_shape=jax.ShapeDtypeStruct((_H, 4 * _HDIM), jnp.int32),
    )(tt, tt, tt, tt)


def _sc_gather_quads(tableq, idx3):
    mesh = plsc.VectorSubcoreMesh(core_axis_name="c", subcore_axis_name="s")

    @functools.partial(
        pl.kernel,
        mesh=mesh,
        out_type=jax.ShapeDtypeStruct((_NW, _BPW, 4 * _HDIM), jnp.int32),
        scratch_types=[
            pltpu.VMEM((_NCHUNK, _CHUNK), jnp.int32),
            pltpu.VMEM((_BPW, 4 * _HDIM), jnp.int32),
            pltpu.SemaphoreType.DMA,
        ],
    )
    def k(table_hbm, idx_hbm, out_hbm, idx_v, rows_v, sem):
        wid = lax.axis_index("s") * _NC + lax.axis_index("c")
        pltpu.sync_copy(idx_hbm.at[wid], idx_v)
        copies = [
            pltpu.async_copy(
                table_hbm.at[idx_v.at[j]],
                rows_v.at[pl.ds(j * _CHUNK, _CHUNK)],
                sem,
            )
            for j in range(_NCHUNK)
        ]
        for c in copies:
            c.wait()
        pltpu.sync_copy(rows_v, out_hbm.at[wid])

    return k(tableq, idx3)


def _mm_body(embq_ref, sel_ref, w_ref, b_ref, out_ref):
    q = embq_ref[...]
    sel = sel_ref[...]
    p01 = jnp.where(sel == 0, q[:, 0 * _HDIM:1 * _HDIM],
                    q[:, 1 * _HDIM:2 * _HDIM])
    p23 = jnp.where(sel == 2, q[:, 2 * _HDIM:3 * _HDIM],
                    q[:, 3 * _HDIM:4 * _HDIM])
    picked = lax.bitcast_convert_type(
        jnp.where(sel < 2, p01, p23), jnp.uint32
    )
    lo = lax.bitcast_convert_type(
        (picked & 0xFFFF).astype(jnp.uint16), jnp.bfloat16
    )
    hi = lax.bitcast_convert_type(
        (picked >> 16).astype(jnp.uint16), jnp.bfloat16
    )
    emb = jnp.concatenate([lo, hi], axis=1)
    out_ref[...] = (
        lax.dot_general(
            w_ref[...], emb, (((1,), (1,)), ((), ())),
            preferred_element_type=jnp.float32,
        )
        + b_ref[...]
    )


def _tc_matmul_t(embq, sel, wb, b2):
    bb = 2048
    grid = (B // bb,)
    return pl.pallas_call(
        _mm_body,
        grid=grid,
        in_specs=[
            pl.BlockSpec((bb, 4 * _HDIM), lambda i: (i, 0)),
            pl.BlockSpec((bb, 1), lambda i: (i, 0)),
            pl.BlockSpec((N_OUT, DIM), lambda i: (0, 0)),
            pl.BlockSpec((N_OUT, 1), lambda i: (0, 0)),
        ],
        out_specs=pl.BlockSpec((N_OUT, bb), lambda i: (0, i)),
        out_shape=jax.ShapeDtypeStruct((N_OUT, B), jnp.float32),
    )(embq, sel, wb, b2)


def kernel(x, table, W, b):
    xi = x.astype(jnp.int32)
    tableq = _tc_convert(table.T)
    sel = xi // _H
    q = xi - sel * _H
    idx3 = q.reshape(_NW, _NCHUNK, _CHUNK)
    sel2 = sel.reshape(B, 1)
    embq = _sc_gather_quads(tableq, idx3).reshape(B, 4 * _HDIM)
    wb = W.astype(jnp.bfloat16)
    b2 = b.reshape(N_OUT, 1)
    out_t = _tc_matmul_t(embq, sel2, wb, b2)
    return out_t.T


# speedup vs baseline: 1.6332x; 1.6332x over previous
import functools

import jax
import jax.numpy as jnp
from jax import lax
from jax.experimental import pallas as pl
from jax.experimental.pallas import tpu as pltpu
from jax.experimental.pallas import tpu_sc as plsc

B = 16384
DIM = 64
N_OUT = 1000

_NC = 2
_NS = 16
_NW = _NC * _NS
_BPW = B // _NW
_CHUNK = 128
_NCHUNK = _BPW // _CHUNK

_VB = 8192
_H = 31 * _VB
_HDIM = DIM // 2


def _conv_body(t0_ref, t1_ref, t2_ref, t3_ref, out_ref):
    eye = (
        lax.broadcasted_iota(jnp.int32, (DIM, DIM), 0)
        == lax.broadcasted_iota(jnp.int32, (DIM, DIM), 1)
    ).astype(jnp.bfloat16)
    dims = (((0,), (0,)), ((), ()))

    def tpart(ref):
        t = lax.dot_general(
            ref[...].astype(jnp.bfloat16), eye, dims,
            preferred_element_type=jnp.float32,
        )
        return lax.bitcast_convert_type(t, jnp.uint32)

    u0, u1, u2, u3 = tpart(t0_ref), tpart(t1_ref), tpart(t2_ref), tpart(t3_ref)
    hi = jnp.uint32(0xFFFF0000)
    out_ref[:, :DIM] = lax.bitcast_convert_type(
        (u0 >> 16) | (u1 & hi), jnp.int32
    )
    out_ref[:, DIM:] = lax.bitcast_convert_type(
        (u2 >> 16) | (u3 & hi), jnp.int32
    )


def _tc_convert(tt):
    grid = (31,)
    return pl.pallas_call(
        _conv_body,
        grid=grid,
        in_specs=[
            pl.BlockSpec((DIM, _VB), lambda i: (0, i)),
            pl.BlockSpec((DIM, _VB), lambda i: (0, i + 31)),
            pl.BlockSpec((DIM, _VB), lambda i: (0, i + 62)),
            pl.BlockSpec((DIM, _VB), lambda i: (0, jnp.minimum(i + 93, 122))),
        ],
        out_specs=pl.BlockSpec((_VB, 4 * _HDIM), lambda i: (i, 0)),
        out_shape=jax.ShapeDtypeStruct((_H, 4 * _HDIM), jnp.int32),
    )(tt, tt, tt, tt)


def _sc_gather_quads(tableq, idx3):
    mesh = plsc.VectorSubcoreMesh(core_axis_name="c", subcore_axis_name="s")

    @functools.partial(
        pl.kernel,
        mesh=mesh,
        out_type=jax.ShapeDtypeStruct((_NW, _BPW, 4 * _HDIM), jnp.int32),
        scratch_types=[
            pltpu.VMEM((_NCHUNK, _CHUNK), jnp.int32),
            pltpu.VMEM((_BPW, 4 * _HDIM), jnp.int32),
            pltpu.SemaphoreType.DMA,
        ],
    )
    def k(table_hbm, idx_hbm, out_hbm, idx_v, rows_v, sem):
        wid = lax.axis_index("s") * _NC + lax.axis_index("c")
        pltpu.sync_copy(idx_hbm.at[wid], idx_v)
        copies = [
            pltpu.async_copy(
                table_hbm.at[idx_v.at[j]],
                rows_v.at[pl.ds(j * _CHUNK, _CHUNK)],
                sem,
            )
            for j in range(_NCHUNK)
        ]
        for c in copies:
            c.wait()
        pltpu.sync_copy(rows_v, out_hbm.at[wid])

    return k(tableq, idx3)


def _mm_body(embq_ref, sel_ref, w_ref, b_ref, out_ref):
    q = embq_ref[...]
    sel = sel_ref[...]
    chosen = lax.bitcast_convert_type(
        jnp.where(sel < 2, q[:, :DIM], q[:, DIM:]), jnp.uint32
    )
    vlo = lax.bitcast_convert_type(chosen << 16, jnp.float32)
    vhi = lax.bitcast_convert_type(
        chosen & jnp.uint32(0xFFFF0000), jnp.float32
    )
    emb = jnp.where((sel & 1) == 1, vhi, vlo).astype(jnp.bfloat16)
    out_ref[...] = (
        lax.dot_general(
            w_ref[...], emb, (((1,), (1,)), ((), ())),
            preferred_element_type=jnp.float32,
        )
        + b_ref[...]
    )


def _tc_matmul_t(embq, sel, wb, b2):
    bb = 2048
    grid = (B // bb,)
    return pl.pallas_call(
        _mm_body,
        grid=grid,
        in_specs=[
            pl.BlockSpec((bb, 4 * _HDIM), lambda i: (i, 0)),
            pl.BlockSpec((bb, 1), lambda i: (i, 0)),
            pl.BlockSpec((N_OUT, DIM), lambda i: (0, 0)),
            pl.BlockSpec((N_OUT, 1), lambda i: (0, 0)),
        ],
        out_specs=pl.BlockSpec((N_OUT, bb), lambda i: (0, i)),
        out_shape=jax.ShapeDtypeStruct((N_OUT, B), jnp.float32),
    )(embq, sel, wb, b2)


def kernel(x, table, W, b):
    xi = x.astype(jnp.int32)
    tableq = _tc_convert(table.T)
    sel = xi // _H
    q = xi - sel * _H
    idx3 = q.reshape(_NW, _NCHUNK, _CHUNK)
    sel2 = sel.reshape(B, 1)
    embq = _sc_gather_quads(tableq, idx3).reshape(B, 4 * _HDIM)
    wb = W.astype(jnp.bfloat16)
    b2 = b.reshape(N_OUT, 1)
    out_t = _tc_matmul_t(embq, sel2, wb, b2)
    return out_t.T


# speedup vs baseline: 1.6711x; 1.0232x over previous
import functools

import jax
import jax.numpy as jnp
from jax import lax
from jax.experimental import pallas as pl
from jax.experimental.pallas import tpu as pltpu
from jax.experimental.pallas import tpu_sc as plsc

B = 16384
DIM = 64
N_OUT = 1000

_NC = 2
_NS = 16
_NW = _NC * _NS
_BPW = B // _NW
_CHUNK = 128
_NCHUNK = _BPW // _CHUNK

_VB = 16384
_NB = 16
_H = _NB * _VB
_HDIM = DIM // 2


def _conv_body(t0_ref, t1_ref, t2_ref, t3_ref, out_ref):
    eye = (
        lax.broadcasted_iota(jnp.int32, (DIM, DIM), 0)
        == lax.broadcasted_iota(jnp.int32, (DIM, DIM), 1)
    ).astype(jnp.bfloat16)
    dims = (((0,), (0,)), ((), ()))

    def tpart(ref):
        t = lax.dot_general(
            ref[...].astype(jnp.bfloat16), eye, dims,
            preferred_element_type=jnp.float32,
        )
        return lax.bitcast_convert_type(t, jnp.uint32)

    u0, u1, u2, u3 = tpart(t0_ref), tpart(t1_ref), tpart(t2_ref), tpart(t3_ref)
    hi = jnp.uint32(0xFFFF0000)
    out_ref[:, :DIM] = lax.bitcast_convert_type(
        (u0 >> 16) | (u1 & hi), jnp.int32
    )
    out_ref[:, DIM:] = lax.bitcast_convert_type(
        (u2 >> 16) | (u3 & hi), jnp.int32
    )


def _tc_convert(tt):
    grid = (_NB,)
    last = 1000000 // _VB
    return pl.pallas_call(
        _conv_body,
        grid=grid,
        in_specs=[
            pl.BlockSpec((DIM, _VB), lambda i: (0, i)),
            pl.BlockSpec((DIM, _VB), lambda i: (0, i + _NB)),
            pl.BlockSpec((DIM, _VB), lambda i: (0, i + 2 * _NB)),
            pl.BlockSpec(
                (DIM, _VB), lambda i: (0, jnp.minimum(i + 3 * _NB, last))
            ),
        ],
        out_specs=pl.BlockSpec((_VB, 4 * _HDIM), lambda i: (i, 0)),
        out_shape=jax.ShapeDtypeStruct((_H, 4 * _HDIM), jnp.int32),
    )(tt, tt, tt, tt)


def _sc_gather_quads(tableq, idx3):
    mesh = plsc.VectorSubcoreMesh(core_axis_name="c", subcore_axis_name="s")

    @functools.partial(
        pl.kernel,
        mesh=mesh,
        out_type=jax.ShapeDtypeStruct((_NW, _BPW, 4 * _HDIM), jnp.int32),
        scratch_types=[
            pltpu.VMEM((_NCHUNK, _CHUNK), jnp.int32),
            pltpu.VMEM((_BPW, 4 * _HDIM), jnp.int32),
            pltpu.SemaphoreType.DMA,
        ],
    )
    def k(table_hbm, idx_hbm, out_hbm, idx_v, rows_v, sem):
        wid = lax.axis_index("s") * _NC + lax.axis_index("c")
        pltpu.sync_copy(idx_hbm.at[wid], idx_v)
        copies = [
            pltpu.async_copy(
                table_hbm.at[idx_v.at[j]],
                rows_v.at[pl.ds(j * _CHUNK, _CHUNK)],
                sem,
            )
            for j in range(_NCHUNK)
        ]
        for c in copies:
            c.wait()
        pltpu.sync_copy(rows_v, out_hbm.at[wid])

    return k(tableq, idx3)


def _mm_body(embq_ref, sel_ref, w_ref, b_ref, out_ref):
    q = embq_ref[...]
    sel = sel_ref[...]
    chosen = lax.bitcast_convert_type(
        jnp.where(sel < 2, q[:, :DIM], q[:, DIM:]), jnp.uint32
    )
    vlo = lax.bitcast_convert_type(chosen << 16, jnp.float32)
    vhi = lax.bitcast_convert_type(
        chosen & jnp.uint32(0xFFFF0000), jnp.float32
    )
    emb = jnp.where((sel & 1) == 1, vhi, vlo).astype(jnp.bfloat16)
    out_ref[...] = (
        lax.dot_general(
            w_ref[...], emb, (((1,), (1,)), ((), ())),
            preferred_element_type=jnp.float32,
        )
        + b_ref[...]
    )


def _tc_matmul_t(embq, sel, wb, b2):
    bb = 4096
    grid = (B // bb,)
    return pl.pallas_call(
        _mm_body,
        grid=grid,
        in_specs=[
            pl.BlockSpec((bb, 4 * _HDIM), lambda i: (i, 0)),
            pl.BlockSpec((bb, 1), lambda i: (i, 0)),
            pl.BlockSpec((N_OUT, DIM), lambda i: (0, 0)),
            pl.BlockSpec((N_OUT, 1), lambda i: (0, 0)),
        ],
        out_specs=pl.BlockSpec((N_OUT, bb), lambda i: (0, i)),
        out_shape=jax.ShapeDtypeStruct((N_OUT, B), jnp.float32),
    )(embq, sel, wb, b2)


def kernel(x, table, W, b):
    xi = x.astype(jnp.int32)
    tableq = _tc_convert(table.T)
    sel = xi // _H
    q = xi - sel * _H
    idx3 = q.reshape(_NW, _NCHUNK, _CHUNK)
    sel2 = sel.reshape(B, 1)
    embq = _sc_gather_quads(tableq, idx3).reshape(B, 4 * _HDIM)
    wb = W.astype(jnp.bfloat16)
    b2 = b.reshape(N_OUT, 1)
    out_t = _tc_matmul_t(embq, sel2, wb, b2)
    return out_t.T
